# Initial kernel scaffold; baseline (speedup 1.0000x reference)
import jax
import jax.numpy as jnp
from jax.experimental import pallas as pl


def kernel(x, edge_index, e, W1, b1, gamma, beta, W2, b2):
    raise NotImplementedError("write your pallas kernel here")


# speedup vs baseline: 1.0391x; 1.0391x over previous
import functools

import jax
import jax.numpy as jnp
from jax import lax
from jax.experimental import pallas as pl
from jax.experimental.pallas import tpu as pltpu
from jax.experimental.pallas import tpu_sc as plsc

_HI = lax.Precision.HIGHEST

_N = 10000
_E = 320000
_D = 128
_DE = 4
_EMB = _D + _DE
_NC = 2
_NS = 16
_NW = _NC * _NS
_CHA = 40
_EPTA = _E // _NW
_NCHA = _EPTA // _CHA
_CHB = 80
_EPTB = _E // _NS
_NCHB = _EPTB // _CHB
_RPT = 624
_BN = 2000
_BE = 4000


def _dotT(a, b):
    return lax.dot_general(a, b, (((0,), (0,)), ((), ())),
                           preferred_element_type=jnp.float32, precision=_HI)


def _dot(a, b):
    return lax.dot_general(a, b, (((1,), (0,)), ((), ())),
                           preferred_element_type=jnp.float32, precision=_HI)


def _ee_gram_body(e_ref, ee_ref):
    i = pl.program_id(0)

    @pl.when(i == 0)
    def _():
        ee_ref[...] = jnp.zeros_like(ee_ref)

    eb = e_ref[...]
    ee_ref[...] += jnp.pad(_dotT(eb, eb), ((0, 12), (0, 12)))


def _ee_gram(e):
    return pl.pallas_call(
        _ee_gram_body,
        grid=(_E // _BE,),
        in_specs=[pl.BlockSpec((_BE, _DE), lambda i: (i, 0))],
        out_specs=pl.BlockSpec((16, 16), lambda i: (0, 0)),
        out_shape=jax.ShapeDtypeStruct((16, 16), jnp.float32),
    )(e)


def _stats_body(x_ref, sa0_ref, sa1_ref, mx_ref, mct_ref, sv_ref):
    i = pl.program_id(0)

    @pl.when(i == 0)
    def _():
        mx_ref[...] = jnp.zeros_like(mx_ref)
        mct_ref[...] = jnp.zeros_like(mct_ref)
        sv_ref[...] = jnp.zeros_like(sv_ref)

    xb = x_ref[...]
    sa = sa0_ref[...] + sa1_ref[...]
    sa16 = sa[:, :16]
    c = sa[:, 4:5]
    mx_ref[...] += _dotT(xb, xb * c)
    mct_ref[...] += _dotT(sa16, xb)
    sv_ref[...] += jnp.pad(jnp.sum(sa16, axis=0, keepdims=True),
                           ((0, 7), (0, 0)))


def _stats(x, sa0, sa1):
    return pl.pallas_call(
        _stats_body,
        grid=(_N // _BN,),
        in_specs=[
            pl.BlockSpec((_BN, _D), lambda i: (i, 0)),
            pl.BlockSpec((_BN, _D), lambda i: (i, 0)),
            pl.BlockSpec((_BN, _D), lambda i: (i, 0)),
        ],
        out_specs=[
            pl.BlockSpec((_D, _D), lambda i: (0, 0)),
            pl.BlockSpec((16, _D), lambda i: (0, 0)),
            pl.BlockSpec((8, 16), lambda i: (0, 0)),
        ],
        out_shape=[
            jax.ShapeDtypeStruct((_D, _D), jnp.float32),
            jax.ShapeDtypeStruct((16, _D), jnp.float32),
            jax.ShapeDtypeStruct((8, 16), jnp.float32),
        ],
    )(x, sa0, sa1)


def _fold_body(mx_ref, mct_ref, sv_ref, ee_ref, w1_ref, b1_ref, g_ref, be_ref,
               fx_ref, fe_ref):
    w1 = w1_ref[...]
    w1x = w1[:_D, :]
    w1e = w1[_D:, :]
    mx = mx_ref[...]
    mct = mct_ref[...]
    crossT = mct[:4, :]
    sx = mct[4:5, :]
    se = sv_ref[0:1, :4]
    ee = ee_ref[:4, :4]
    mu = jnp.concatenate([sx, se], axis=1) * (1.0 / _E)

    t1 = jnp.sum(w1x * _dot(mx, w1x), axis=0, keepdims=True)
    t2 = 2.0 * jnp.sum(w1e * _dot(crossT, w1x), axis=0, keepdims=True)
    t3 = jnp.sum(w1e * _dot(ee, w1e), axis=0, keepdims=True)
    mean0 = _dot(mu, w1)
    var = (t1 + t2 + t3) * (1.0 / _E) - mean0 * mean0
    mean = mean0 + b1_ref[...]
    scale = g_ref[...] * lax.rsqrt(var + 1e-5)
    w1f = w1 * scale
    b1f = (b1_ref[...] - mean) * scale + be_ref[...]

    fx_ref[...] = jnp.concatenate(
        [w1f[:_D, :_D], w1f[:_D, _DE:]], axis=0).reshape(2, _D, _D)
    felo = jnp.concatenate(
        [w1f[_D:, :_D], b1f[:, :_D], jnp.zeros((3, _D), jnp.float32)], axis=0)
    fehi = jnp.concatenate(
        [w1f[_D:, _DE:], b1f[:, _DE:], jnp.zeros((3, _D), jnp.float32)],
        axis=0)
    fe_ref[...] = jnp.concatenate([felo, fehi], axis=0).reshape(2, 8, _D)


def _fold(mx, mct, sv, ee, w1, b1r, gr, ber):
    return pl.pallas_call(
        _fold_body,
        out_shape=[
            jax.ShapeDtypeStruct((2, _D, _D), jnp.float32),
            jax.ShapeDtypeStruct((2, 8, _D), jnp.float32),
        ],
    )(mx, mct, sv, ee, w1, b1r, gr, ber)


def _xw_body(x_ref, fx_ref, o_ref):
    o_ref[...] = _dot(x_ref[...], fx_ref[0])[None]


def _xw(x, fx):
    return pl.pallas_call(
        _xw_body,
        grid=(2, _N // _BN),
        in_specs=[
            pl.BlockSpec((_BN, _D), lambda w, i: (i, 0)),
            pl.BlockSpec((1, _D, _D), lambda w, i: (w, 0, 0)),
        ],
        out_specs=pl.BlockSpec((1, _BN, _D), lambda w, i: (w, i, 0)),
        out_shape=jax.ShapeDtypeStruct((2, _N, _D), jnp.float32),
    )(x, fx)


def _ew_body(e_ref, fe_ref, o_ref):
    f = fe_ref[0]
    o_ref[...] = (_dot(e_ref[...], f[0:4, :]) + f[4:5, :])[None]


def _ew(e, fe):
    return pl.pallas_call(
        _ew_body,
        grid=(2, _E // _BE),
        in_specs=[
            pl.BlockSpec((_BE, _DE), lambda w, i: (i, 0)),
            pl.BlockSpec((1, 8, _D), lambda w, i: (w, 0, 0)),
        ],
        out_specs=pl.BlockSpec((1, _BE, _D), lambda w, i: (w, i, 0)),
        out_shape=jax.ShapeDtypeStruct((2, _E, _D), jnp.float32),
    )(e, fe)


def _make_src_stats():
    mesh = plsc.VectorSubcoreMesh(core_axis_name="c", subcore_axis_name="s")

    @functools.partial(
        pl.kernel, mesh=mesh,
        out_type=jax.ShapeDtypeStruct((_NC, _N, _D), jnp.float32),
        scratch_types=[
            pltpu.VMEM((2 * _CHA,), jnp.int32),
            pltpu.VMEM((2 * _CHA, _D), jnp.float32),
            pltpu.VMEM((8, _D), jnp.float32),
            pltpu.VMEM_SHARED((_N, _D), jnp.float32),
        ],
    )
    def k(src_hbm, dst_hbm, e128_hbm, zeros_hbm, consts_hbm, out_hbm, idxv,
          stag, cvbuf, acc):
        c = lax.axis_index("c")
        s = lax.axis_index("s")
        wid = c * _NS + s

        pltpu.sync_copy(consts_hbm, cvbuf)

        def irow(i, carry):
            for r in range(_D // 16):
                sl = pl.ds(r * 16, 16)
                stag[_CHA + i, sl] = cvbuf[1, sl]
            return carry

        lax.fori_loop(0, _CHA, irow, 0)
        pltpu.sync_copy(zeros_hbm.at[pl.ds(s * _RPT, _RPT), :],
                        acc.at[pl.ds(s * _RPT, _RPT), :])

        @pl.when(s == _NS - 1)
        def _():
            pltpu.sync_copy(zeros_hbm.at[pl.ds(_NS * _RPT, 16), :],
                            acc.at[pl.ds(_NS * _RPT, 16), :])

        plsc.subcore_barrier()

        base0 = wid * _EPTA

        def body(i, carry):
            base = base0 + i * _CHA
            pltpu.sync_copy(src_hbm.at[pl.ds(base, _CHA)],
                            idxv.at[pl.ds(0, _CHA)])
            pltpu.sync_copy(dst_hbm.at[pl.ds(base, _CHA)],
                            idxv.at[pl.ds(_CHA, _CHA)])
            pltpu.sync_copy(e128_hbm.at[pl.ds(base, _CHA), :],
                            stag.at[pl.ds(0, _CHA), :])
            pltpu.sync_copy(stag, acc.at[idxv], add=True)
            return carry

        lax.fori_loop(0, _NCHA, body, 0)
        plsc.subcore_barrier()
        pltpu.sync_copy(acc.at[pl.ds(s * _RPT, _RPT), :],
                        out_hbm.at[c, pl.ds(s * _RPT, _RPT), :])

        @pl.when(s == _NS - 1)
        def _():
            pltpu.sync_copy(acc.at[pl.ds(_NS * _RPT, 16), :],
                            out_hbm.at[c, pl.ds(_NS * _RPT, 16), :])

    return k


def _make_edge_pass():
    mesh = plsc.VectorSubcoreMesh(core_axis_name="c", subcore_axis_name="s")

    @functools.partial(
        pl.kernel, mesh=mesh,
        out_type=jax.ShapeDtypeStruct((_NC, _N, _D), jnp.float32),
        scratch_types=[
            pltpu.VMEM((_CHB,), jnp.int32),
            pltpu.VMEM((_CHB,), jnp.int32),
            pltpu.VMEM((_CHB, _D), jnp.float32),
            pltpu.VMEM((_CHB, _D), jnp.float32),
            pltpu.VMEM_SHARED((_N, _D), jnp.float32),
            pltpu.SemaphoreType.DMA,
        ],
    )
    def k(src_hbm, dst_hbm, xw2_hbm, ew2_hbm, zeros_hbm, out_hbm,
          srcv, dstv, gbuf, wbuf, acc, sem):
        c = lax.axis_index("c")
        s = lax.axis_index("s")

        pltpu.sync_copy(zeros_hbm.at[pl.ds(s * _RPT, _RPT), :],
                        acc.at[pl.ds(s * _RPT, _RPT), :])

        @pl.when(s == _NS - 1)
        def _():
            pltpu.sync_copy(zeros_hbm.at[pl.ds(_NS * _RPT, 16), :],
                            acc.at[pl.ds(_NS * _RPT, 16), :])

        plsc.subcore_barrier()

        base0 = s * _EPTB
        coff = c * _N

        def body(i, carry):
            base = base0 + i * _CHB
            pltpu.sync_copy(src_hbm.at[pl.ds(base, _CHB)], srcv)
            pltpu.sync_copy(dst_hbm.at[pl.ds(base, _CHB)], dstv)
            coffv = jnp.full((16,), coff, jnp.int32)
            for b in range(_CHB // 16):
                sl = pl.ds(b * 16, 16)
                srcv[sl] = srcv[sl] + coffv
            pltpu.async_copy(xw2_hbm.at[srcv], gbuf, sem).wait()
            pltpu.sync_copy(ew2_hbm.at[c, pl.ds(base, _CHB), :], wbuf)

            zf16 = jnp.zeros((16,), jnp.float32)

            def erow(j, carry2):
                for r in range(_D // 16):
                    sl = pl.ds(r * 16, 16)
                    wbuf[j, sl] = jnp.maximum(gbuf[j, sl] + wbuf[j, sl], zf16)
                return carry2

            lax.fori_loop(0, _CHB, erow, 0)
            pltpu.sync_copy(wbuf, acc.at[dstv], add=True)
            return carry

        lax.fori_loop(0, _NCHB, body, 0)
        plsc.subcore_barrier()
        pltpu.sync_copy(acc.at[pl.ds(s * _RPT, _RPT), :],
                        out_hbm.at[c, pl.ds(s * _RPT, _RPT), :])

        @pl.when(s == _NS - 1)
        def _():
            pltpu.sync_copy(acc.at[pl.ds(_NS * _RPT, 16), :],
                            out_hbm.at[c, pl.ds(_NS * _RPT, 16), :])

    return k


def _finish_body(lo_ref, hi_ref, sa0_ref, sa1_ref, x_ref, w2_ref, b2_ref,
                 o_ref):
    lo = lo_ref[...]
    hi = hi_ref[...]
    agg = jnp.concatenate(
        [lo[:, :_DE], hi, jnp.zeros((lo.shape[0], 4), jnp.float32)], axis=1)
    cnt = (sa0_ref[...] + sa1_ref[...])[:, 5:6]
    h = _dot(agg, w2_ref[...]) + cnt * b2_ref[...]
    o_ref[...] = jnp.where(cnt > 0.5, h, x_ref[...])


def _finish(lo, hi, sa0, sa1, x, w2p, b2r):
    return pl.pallas_call(
        _finish_body,
        grid=(_N // _BN,),
        in_specs=[
            pl.BlockSpec((_BN, _D), lambda i: (i, 0)),
            pl.BlockSpec((_BN, _D), lambda i: (i, 0)),
            pl.BlockSpec((_BN, _D), lambda i: (i, 0)),
            pl.BlockSpec((_BN, _D), lambda i: (i, 0)),
            pl.BlockSpec((_BN, _D), lambda i: (i, 0)),
            pl.BlockSpec((136, _D), lambda i: (0, 0)),
            pl.BlockSpec((1, _D), lambda i: (0, 0)),
        ],
        out_specs=pl.BlockSpec((_BN, _D), lambda i: (i, 0)),
        out_shape=jax.ShapeDtypeStruct((_N, _D), jnp.float32),
    )(lo, hi, sa0, sa1, x, w2p, b2r)


def kernel(x, edge_index, e, W1, b1, gamma, beta, W2, b2):
    src = edge_index[0]
    dst = edge_index[1]
    zeros128 = jnp.zeros((_N, _D), jnp.float32)
    consts = jnp.zeros((8, _D), jnp.float32).at[1, 5].set(1.0)
    e128 = jnp.concatenate(
        [e, jnp.ones((_E, 1), jnp.float32),
         jnp.zeros((_E, _D - _DE - 1), jnp.float32)], axis=1)
    ee = _ee_gram(e)
    srcagg = _make_src_stats()(src, dst, e128, zeros128, consts)
    mx, mct, sv = _stats(x, srcagg[0], srcagg[1])
    fx, fe = _fold(mx, mct, sv, ee, W1, b1.reshape(1, -1),
                   gamma.reshape(1, -1), beta.reshape(1, -1))
    xw2 = _xw(x, fx).reshape(2 * _N, _D)
    ew2 = _ew(e, fe)
    aggw = _make_edge_pass()(src, dst, xw2, ew2, zeros128)
    w2p = jnp.pad(W2, ((0, 4), (0, 0)))
    return _finish(aggw[0], aggw[1], srcagg[0], srcagg[1], x, w2p,
                   b2.reshape(1, -1))


# speedup vs baseline: 1.4374x; 1.3833x over previous
import functools

import jax
import jax.numpy as jnp
from jax import lax
from jax.experimental import pallas as pl
from jax.experimental.pallas import tpu as pltpu
from jax.experimental.pallas import tpu_sc as plsc

_HI = lax.Precision.HIGHEST

_N = 10000
_E = 320000
_D = 128
_DE = 4
_EMB = _D + _DE
_NC = 2
_NS = 16
_NW = _NC * _NS
_CHA = 40
_EPTA = _E // _NW
_NCHA = _EPTA // _CHA
_CHB = 80
_EPTB = _E // _NS
_NCHB = _EPTB // _CHB
_RPT = 624
_BN = 2000
_BE = 4000


def _dotT(a, b):
    return lax.dot_general(a, b, (((0,), (0,)), ((), ())),
                           preferred_element_type=jnp.float32, precision=_HI)


def _dot(a, b):
    return lax.dot_general(a, b, (((1,), (0,)), ((), ())),
                           preferred_element_type=jnp.float32, precision=_HI)


def _ee_gram_body(e_ref, ee_ref):
    i = pl.program_id(0)

    @pl.when(i == 0)
    def _():
        ee_ref[...] = jnp.zeros_like(ee_ref)

    eb = e_ref[...]
    ee_ref[...] += jnp.pad(_dotT(eb, eb), ((0, 12), (0, 12)))


def _ee_gram(e):
    return pl.pallas_call(
        _ee_gram_body,
        grid=(_E // _BE,),
        in_specs=[pl.BlockSpec((_BE, _DE), lambda i: (i, 0))],
        out_specs=pl.BlockSpec((16, 16), lambda i: (0, 0)),
        out_shape=jax.ShapeDtypeStruct((16, 16), jnp.float32),
    )(e)


def _stats_body(x_ref, sa0_ref, sa1_ref, mx_ref, mct_ref, sv_ref):
    i = pl.program_id(0)

    @pl.when(i == 0)
    def _():
        mx_ref[...] = jnp.zeros_like(mx_ref)
        mct_ref[...] = jnp.zeros_like(mct_ref)
        sv_ref[...] = jnp.zeros_like(sv_ref)

    xb = x_ref[...]
    sa = sa0_ref[...] + sa1_ref[...]
    sa16 = sa[:, :16]
    c = sa[:, 4:5]
    mx_ref[...] += _dotT(xb, xb * c)
    mct_ref[...] += _dotT(sa16, xb)
    sv_ref[...] += jnp.pad(jnp.sum(sa16, axis=0, keepdims=True),
                           ((0, 7), (0, 0)))


def _stats(x, sa0, sa1):
    return pl.pallas_call(
        _stats_body,
        grid=(_N // _BN,),
        in_specs=[
            pl.BlockSpec((_BN, _D), lambda i: (i, 0)),
            pl.BlockSpec((_BN, _D), lambda i: (i, 0)),
            pl.BlockSpec((_BN, _D), lambda i: (i, 0)),
        ],
        out_specs=[
            pl.BlockSpec((_D, _D), lambda i: (0, 0)),
            pl.BlockSpec((16, _D), lambda i: (0, 0)),
            pl.BlockSpec((8, 16), lambda i: (0, 0)),
        ],
        out_shape=[
            jax.ShapeDtypeStruct((_D, _D), jnp.float32),
            jax.ShapeDtypeStruct((16, _D), jnp.float32),
            jax.ShapeDtypeStruct((8, 16), jnp.float32),
        ],
    )(x, sa0, sa1)


def _fold_body(mx_ref, mct_ref, sv_ref, ee_ref, w1_ref, b1_ref, g_ref, be_ref,
               fx_ref, fe_ref):
    w1 = w1_ref[...]
    w1x = w1[:_D, :]
    w1e = w1[_D:, :]
    mx = mx_ref[...]
    mct = mct_ref[...]
    crossT = mct[:4, :]
    sx = mct[4:5, :]
    se = sv_ref[0:1, :4]
    ee = ee_ref[:4, :4]
    mu = jnp.concatenate([sx, se], axis=1) * (1.0 / _E)

    t1 = jnp.sum(w1x * _dot(mx, w1x), axis=0, keepdims=True)
    t2 = 2.0 * jnp.sum(w1e * _dot(crossT, w1x), axis=0, keepdims=True)
    t3 = jnp.sum(w1e * _dot(ee, w1e), axis=0, keepdims=True)
    mean0 = _dot(mu, w1)
    var = (t1 + t2 + t3) * (1.0 / _E) - mean0 * mean0
    mean = mean0 + b1_ref[...]
    scale = g_ref[...] * lax.rsqrt(var + 1e-5)
    w1f = w1 * scale
    b1f = (b1_ref[...] - mean) * scale + be_ref[...]

    fx_ref[...] = jnp.concatenate(
        [w1f[:_D, :_D], w1f[:_D, _DE:]], axis=0).reshape(2, _D, _D)
    felo = jnp.concatenate(
        [w1f[_D:, :_D], b1f[:, :_D], jnp.zeros((3, _D), jnp.float32)], axis=0)
    fehi = jnp.concatenate(
        [w1f[_D:, _DE:], b1f[:, _DE:], jnp.zeros((3, _D), jnp.float32)],
        axis=0)
    fe_ref[...] = jnp.concatenate([felo, fehi], axis=0).reshape(2, 8, _D)


def _fold(mx, mct, sv, ee, w1, b1r, gr, ber):
    return pl.pallas_call(
        _fold_body,
        out_shape=[
            jax.ShapeDtypeStruct((2, _D, _D), jnp.float32),
            jax.ShapeDtypeStruct((2, 8, _D), jnp.float32),
        ],
    )(mx, mct, sv, ee, w1, b1r, gr, ber)


def _xw_body(x_ref, fx_ref, o_ref):
    o_ref[...] = _dot(x_ref[...], fx_ref[0])[None]


def _xw(x, fx):
    return pl.pallas_call(
        _xw_body,
        grid=(2, _N // _BN),
        in_specs=[
            pl.BlockSpec((_BN, _D), lambda w, i: (i, 0)),
            pl.BlockSpec((1, _D, _D), lambda w, i: (w, 0, 0)),
        ],
        out_specs=pl.BlockSpec((1, _BN, _D), lambda w, i: (w, i, 0)),
        out_shape=jax.ShapeDtypeStruct((2, _N, _D), jnp.float32),
    )(x, fx)


def _ew_body(e_ref, fe_ref, o_ref):
    f = fe_ref[0]
    o_ref[...] = (_dot(e_ref[...], f[0:4, :]) + f[4:5, :])[None]


def _ew(e, fe):
    return pl.pallas_call(
        _ew_body,
        grid=(2, _E // _BE),
        in_specs=[
            pl.BlockSpec((_BE, _DE), lambda w, i: (i, 0)),
            pl.BlockSpec((1, 8, _D), lambda w, i: (w, 0, 0)),
        ],
        out_specs=pl.BlockSpec((1, _BE, _D), lambda w, i: (w, i, 0)),
        out_shape=jax.ShapeDtypeStruct((2, _E, _D), jnp.float32),
    )(e, fe)


def _make_src_stats():
    mesh = plsc.VectorSubcoreMesh(core_axis_name="c", subcore_axis_name="s")

    @functools.partial(
        pl.kernel, mesh=mesh,
        out_type=jax.ShapeDtypeStruct((_NC, _N, _D), jnp.float32),
        scratch_types=[
            pltpu.VMEM((2 * _CHA,), jnp.int32),
            pltpu.VMEM((2 * _CHA, _D), jnp.float32),
            pltpu.VMEM((8, _D), jnp.float32),
            pltpu.VMEM_SHARED((_N, _D), jnp.float32),
        ],
    )
    def k(src_hbm, dst_hbm, e128_hbm, zeros_hbm, consts_hbm, out_hbm, idxv,
          stag, cvbuf, acc):
        c = lax.axis_index("c")
        s = lax.axis_index("s")
        wid = c * _NS + s

        pltpu.sync_copy(consts_hbm, cvbuf)

        def irow(i, carry):
            for r in range(_D // 16):
                sl = pl.ds(r * 16, 16)
                stag[_CHA + i, sl] = cvbuf[1, sl]
            return carry

        lax.fori_loop(0, _CHA, irow, 0)
        pltpu.sync_copy(zeros_hbm.at[pl.ds(s * _RPT, _RPT), :],
                        acc.at[pl.ds(s * _RPT, _RPT), :])

        @pl.when(s == _NS - 1)
        def _():
            pltpu.sync_copy(zeros_hbm.at[pl.ds(_NS * _RPT, 16), :],
                            acc.at[pl.ds(_NS * _RPT, 16), :])

        plsc.subcore_barrier()

        base0 = wid * _EPTA

        def body(i, carry):
            base = base0 + i * _CHA
            pltpu.sync_copy(src_hbm.at[pl.ds(base, _CHA)],
                            idxv.at[pl.ds(0, _CHA)])
            pltpu.sync_copy(dst_hbm.at[pl.ds(base, _CHA)],
                            idxv.at[pl.ds(_CHA, _CHA)])
            pltpu.sync_copy(e128_hbm.at[pl.ds(base, _CHA), :],
                            stag.at[pl.ds(0, _CHA), :])
            pltpu.sync_copy(stag, acc.at[idxv], add=True)
            return carry

        lax.fori_loop(0, _NCHA, body, 0)
        plsc.subcore_barrier()
        pltpu.sync_copy(acc.at[pl.ds(s * _RPT, _RPT), :],
                        out_hbm.at[c, pl.ds(s * _RPT, _RPT), :])

        @pl.when(s == _NS - 1)
        def _():
            pltpu.sync_copy(acc.at[pl.ds(_NS * _RPT, 16), :],
                            out_hbm.at[c, pl.ds(_NS * _RPT, 16), :])

    return k


def _make_edge_pass():
    mesh = plsc.VectorSubcoreMesh(core_axis_name="c", subcore_axis_name="s")

    @functools.partial(
        pl.kernel, mesh=mesh,
        out_type=jax.ShapeDtypeStruct((_NC, _N, _D), jnp.float32),
        scratch_types=[
            pltpu.VMEM((_CHB,), jnp.int32),
            pltpu.VMEM((_CHB,), jnp.int32),
            pltpu.VMEM((_CHB, _D), jnp.float32),
            pltpu.VMEM((_CHB, _D), jnp.float32),
            pltpu.VMEM((_CHB,), jnp.int32),
            pltpu.VMEM((_CHB,), jnp.int32),
            pltpu.VMEM((_CHB, _D), jnp.float32),
            pltpu.VMEM((_CHB, _D), jnp.float32),
            pltpu.VMEM_SHARED((_N, _D), jnp.float32),
            pltpu.SemaphoreType.DMA,
            pltpu.SemaphoreType.DMA,
            pltpu.SemaphoreType.DMA,
            pltpu.SemaphoreType.DMA,
            pltpu.SemaphoreType.DMA,
            pltpu.SemaphoreType.DMA,
        ],
    )
    def k(src_hbm, dst_hbm, xw2_hbm, ew2_hbm, zeros_hbm, out_hbm,
          srcva, dstva, gbufa, wbufa, srcvb, dstvb, gbufb, wbufb, acc,
          isema, esema, gsema, isemb, esemb, gsemb):
        c = lax.axis_index("c")
        s = lax.axis_index("s")

        pltpu.sync_copy(zeros_hbm.at[pl.ds(s * _RPT, _RPT), :],
                        acc.at[pl.ds(s * _RPT, _RPT), :])

        @pl.when(s == _NS - 1)
        def _():
            pltpu.sync_copy(zeros_hbm.at[pl.ds(_NS * _RPT, 16), :],
                            acc.at[pl.ds(_NS * _RPT, 16), :])

        plsc.subcore_barrier()

        base0 = s * _EPTB
        coffv = jnp.full((16,), c * _N, jnp.int32)
        zf16 = jnp.zeros((16,), jnp.float32)
        bufs = ((srcva, dstva, gbufa, wbufa, isema, esema, gsema),
                (srcvb, dstvb, gbufb, wbufb, isemb, esemb, gsemb))

        def start_ie(ci, bi):
            srcv, dstv, _, wbuf, isem, esem, _ = bufs[bi]
            base = base0 + ci * _CHB
            pltpu.async_copy(src_hbm.at[pl.ds(base, _CHB)], srcv, isem)
            pltpu.async_copy(dst_hbm.at[pl.ds(base, _CHB)], dstv, isem)
            pltpu.async_copy(ew2_hbm.at[c, pl.ds(base, _CHB), :], wbuf, esem)

        def wait_idx(ci, bi):
            srcv, dstv, _, _, isem, _, _ = bufs[bi]
            base = base0 + ci * _CHB
            pltpu.make_async_copy(src_hbm.at[pl.ds(base, _CHB)], srcv,
                                  isem).wait()
            pltpu.make_async_copy(dst_hbm.at[pl.ds(base, _CHB)], dstv,
                                  isem).wait()

        def start_gather(bi):
            srcv, _, gbuf, _, _, _, gsem = bufs[bi]
            for b in range(_CHB // 16):
                sl = pl.ds(b * 16, 16)
                srcv[sl] = srcv[sl] + coffv
            pltpu.async_copy(xw2_hbm.at[srcv], gbuf, gsem)

        def compute_scatter(ci, bi):
            srcv, dstv, gbuf, wbuf, _, esem, gsem = bufs[bi]
            base = base0 + ci * _CHB
            pltpu.make_async_copy(xw2_hbm.at[srcv], gbuf, gsem).wait()
            pltpu.make_async_copy(ew2_hbm.at[c, pl.ds(base, _CHB), :], wbuf,
                                  esem).wait()

            def erow(j, carry2):
                for jj in range(2):
                    for r in range(_D // 16):
                        sl = pl.ds(r * 16, 16)
                        a = gbuf[2 * j + jj, sl] + wbuf[2 * j + jj, sl]
                        wbuf[2 * j + jj, sl] = jnp.maximum(a, zf16)
                return carry2

            lax.fori_loop(0, _CHB // 2, erow, 0)
            pltpu.sync_copy(wbuf, acc.at[dstv], add=True)

        start_ie(0, 0)
        start_ie(1, 1)
        wait_idx(0, 0)
        start_gather(0)

        npair = _NCHB // 2

        def pair(jj, carry):
            c0 = 2 * jj
            c1 = c0 + 1
            wait_idx(c1, 1)
            start_gather(1)
            compute_scatter(c0, 0)

            @pl.when(jj + 1 < npair)
            def _():
                start_ie(c0 + 2, 0)
                wait_idx(c0 + 2, 0)
                start_gather(0)

            compute_scatter(c1, 1)

            @pl.when(jj + 1 < npair)
            def _():
                start_ie(c1 + 2, 1)

            return carry

        lax.fori_loop(0, npair, pair, 0)
        plsc.subcore_barrier()
        pltpu.sync_copy(acc.at[pl.ds(s * _RPT, _RPT), :],
                        out_hbm.at[c, pl.ds(s * _RPT, _RPT), :])

        @pl.when(s == _NS - 1)
        def _():
            pltpu.sync_copy(acc.at[pl.ds(_NS * _RPT, 16), :],
                            out_hbm.at[c, pl.ds(_NS * _RPT, 16), :])

    return k


def _finish_body(lo_ref, hi_ref, sa0_ref, sa1_ref, x_ref, w2_ref, b2_ref,
                 o_ref):
    lo = lo_ref[...]
    hi = hi_ref[...]
    agg = jnp.concatenate(
        [lo[:, :_DE], hi, jnp.zeros((lo.shape[0], 4), jnp.float32)], axis=1)
    cnt = (sa0_ref[...] + sa1_ref[...])[:, 5:6]
    h = _dot(agg, w2_ref[...]) + cnt * b2_ref[...]
    o_ref[...] = jnp.where(cnt > 0.5, h, x_ref[...])


def _finish(lo, hi, sa0, sa1, x, w2p, b2r):
    return pl.pallas_call(
        _finish_body,
        grid=(_N // _BN,),
        in_specs=[
            pl.BlockSpec((_BN, _D), lambda i: (i, 0)),
            pl.BlockSpec((_BN, _D), lambda i: (i, 0)),
            pl.BlockSpec((_BN, _D), lambda i: (i, 0)),
            pl.BlockSpec((_BN, _D), lambda i: (i, 0)),
            pl.BlockSpec((_BN, _D), lambda i: (i, 0)),
            pl.BlockSpec((136, _D), lambda i: (0, 0)),
            pl.BlockSpec((1, _D), lambda i: (0, 0)),
        ],
        out_specs=pl.BlockSpec((_BN, _D), lambda i: (i, 0)),
        out_shape=jax.ShapeDtypeStruct((_N, _D), jnp.float32),
    )(lo, hi, sa0, sa1, x, w2p, b2r)


def kernel(x, edge_index, e, W1, b1, gamma, beta, W2, b2):
    src = edge_index[0]
    dst = edge_index[1]
    zeros128 = jnp.zeros((_N, _D), jnp.float32)
    consts = jnp.zeros((8, _D), jnp.float32).at[1, 5].set(1.0)
    e128 = jnp.concatenate(
        [e, jnp.ones((_E, 1), jnp.float32),
         jnp.zeros((_E, _D - _DE - 1), jnp.float32)], axis=1)
    ee = _ee_gram(e)
    srcagg = _make_src_stats()(src, dst, e128, zeros128, consts)
    mx, mct, sv = _stats(x, srcagg[0], srcagg[1])
    fx, fe = _fold(mx, mct, sv, ee, W1, b1.reshape(1, -1),
                   gamma.reshape(1, -1), beta.reshape(1, -1))
    xw2 = _xw(x, fx).reshape(2 * _N, _D)
    ew2 = _ew(e, fe)
    aggw = _make_edge_pass()(src, dst, xw2, ew2, zeros128)
    w2p = jnp.pad(W2, ((0, 4), (0, 0)))
    return _finish(aggw[0], aggw[1], srcagg[0], srcagg[1], x, w2p,
                   b2.reshape(1, -1))


# speedup vs baseline: 1.8519x; 1.2883x over previous
import functools

import jax
import jax.numpy as jnp
from jax import lax
from jax.experimental import pallas as pl
from jax.experimental.pallas import tpu as pltpu
from jax.experimental.pallas import tpu_sc as plsc

_HI = lax.Precision.HIGHEST

_N = 10000
_E = 320000
_D = 128
_DE = 4
_EMB = _D + _DE
_NC = 2
_NS = 16
_NW = _NC * _NS
_CHA = 40
_EPTA = _E // _NW
_NCHA = _EPTA // _CHA
_CHB = 80
_EPTB = _E // _NS
_NCHB = _EPTB // _CHB
_RPT = 624
_BN = 2000
_BE = 4000


def _dotT(a, b):
    return lax.dot_general(a, b, (((0,), (0,)), ((), ())),
                           preferred_element_type=jnp.float32, precision=_HI)


def _dot(a, b):
    return lax.dot_general(a, b, (((1,), (0,)), ((), ())),
                           preferred_element_type=jnp.float32, precision=_HI)


def _ee_gram_body(e_ref, ee_ref):
    i = pl.program_id(0)

    @pl.when(i == 0)
    def _():
        ee_ref[...] = jnp.zeros_like(ee_ref)

    eb = e_ref[...]
    ee_ref[...] += jnp.pad(_dotT(eb, eb), ((0, 12), (0, 12)))


def _ee_gram(e):
    return pl.pallas_call(
        _ee_gram_body,
        grid=(_E // _BE,),
        in_specs=[pl.BlockSpec((_BE, _DE), lambda i: (i, 0))],
        out_specs=pl.BlockSpec((16, 16), lambda i: (0, 0)),
        out_shape=jax.ShapeDtypeStruct((16, 16), jnp.float32),
    )(e)


def _stats_body(x_ref, sa0_ref, sa1_ref, mx_ref, mct_ref, sv_ref):
    i = pl.program_id(0)

    @pl.when(i == 0)
    def _():
        mx_ref[...] = jnp.zeros_like(mx_ref)
        mct_ref[...] = jnp.zeros_like(mct_ref)
        sv_ref[...] = jnp.zeros_like(sv_ref)

    xb = x_ref[...]
    sa = sa0_ref[...] + sa1_ref[...]
    sa16 = sa[:, :16]
    c = sa[:, 4:5]
    mx_ref[...] += _dotT(xb, xb * c)
    mct_ref[...] += _dotT(sa16, xb)
    sv_ref[...] += jnp.pad(jnp.sum(sa16, axis=0, keepdims=True),
                           ((0, 7), (0, 0)))


def _stats(x, sa0, sa1):
    return pl.pallas_call(
        _stats_body,
        grid=(_N // _BN,),
        in_specs=[
            pl.BlockSpec((_BN, _D), lambda i: (i, 0)),
            pl.BlockSpec((_BN, _D), lambda i: (i, 0)),
            pl.BlockSpec((_BN, _D), lambda i: (i, 0)),
        ],
        out_specs=[
            pl.BlockSpec((_D, _D), lambda i: (0, 0)),
            pl.BlockSpec((16, _D), lambda i: (0, 0)),
            pl.BlockSpec((8, 16), lambda i: (0, 0)),
        ],
        out_shape=[
            jax.ShapeDtypeStruct((_D, _D), jnp.float32),
            jax.ShapeDtypeStruct((16, _D), jnp.float32),
            jax.ShapeDtypeStruct((8, 16), jnp.float32),
        ],
    )(x, sa0, sa1)


def _fold_body(mx_ref, mct_ref, sv_ref, ee_ref, w1_ref, b1_ref, g_ref, be_ref,
               fx_ref, fe_ref):
    w1 = w1_ref[...]
    w1x = w1[:_D, :]
    w1e = w1[_D:, :]
    mx = mx_ref[...]
    mct = mct_ref[...]
    crossT = mct[:4, :]
    sx = mct[4:5, :]
    se = sv_ref[0:1, :4]
    ee = ee_ref[:4, :4]
    mu = jnp.concatenate([sx, se], axis=1) * (1.0 / _E)

    t1 = jnp.sum(w1x * _dot(mx, w1x), axis=0, keepdims=True)
    t2 = 2.0 * jnp.sum(w1e * _dot(crossT, w1x), axis=0, keepdims=True)
    t3 = jnp.sum(w1e * _dot(ee, w1e), axis=0, keepdims=True)
    mean0 = _dot(mu, w1)
    var = (t1 + t2 + t3) * (1.0 / _E) - mean0 * mean0
    mean = mean0 + b1_ref[...]
    scale = g_ref[...] * lax.rsqrt(var + 1e-5)
    w1f = w1 * scale
    b1f = (b1_ref[...] - mean) * scale + be_ref[...]

    fx_ref[...] = jnp.concatenate(
        [w1f[:_D, :_D], w1f[:_D, _DE:]], axis=0).reshape(2, _D, _D)
    felo = jnp.concatenate(
        [w1f[_D:, :_D], b1f[:, :_D], jnp.zeros((3, _D), jnp.float32)], axis=0)
    fehi = jnp.concatenate(
        [w1f[_D:, _DE:], b1f[:, _DE:], jnp.zeros((3, _D), jnp.float32)],
        axis=0)
    fe_ref[...] = jnp.concatenate([felo, fehi], axis=0).reshape(2, 8, _D)


def _fold(mx, mct, sv, ee, w1, b1r, gr, ber):
    return pl.pallas_call(
        _fold_body,
        out_shape=[
            jax.ShapeDtypeStruct((2, _D, _D), jnp.float32),
            jax.ShapeDtypeStruct((2, 8, _D), jnp.float32),
        ],
    )(mx, mct, sv, ee, w1, b1r, gr, ber)


def _xw_body(x_ref, fx_ref, o_ref):
    o_ref[...] = _dot(x_ref[...], fx_ref[0])[None]


def _xw(x, fx):
    return pl.pallas_call(
        _xw_body,
        grid=(2, _N // _BN),
        in_specs=[
            pl.BlockSpec((_BN, _D), lambda w, i: (i, 0)),
            pl.BlockSpec((1, _D, _D), lambda w, i: (w, 0, 0)),
        ],
        out_specs=pl.BlockSpec((1, _BN, _D), lambda w, i: (w, i, 0)),
        out_shape=jax.ShapeDtypeStruct((2, _N, _D), jnp.float32),
    )(x, fx)


def _ew_body(e_ref, fe_ref, o_ref):
    f = fe_ref[0]
    o_ref[...] = (_dot(e_ref[...], f[0:4, :]) + f[4:5, :])[None]


def _ew(e, fe):
    return pl.pallas_call(
        _ew_body,
        grid=(2, _E // _BE),
        in_specs=[
            pl.BlockSpec((_BE, _DE), lambda w, i: (i, 0)),
            pl.BlockSpec((1, 8, _D), lambda w, i: (w, 0, 0)),
        ],
        out_specs=pl.BlockSpec((1, _BE, _D), lambda w, i: (w, i, 0)),
        out_shape=jax.ShapeDtypeStruct((2, _E, _D), jnp.float32),
    )(e, fe)


def _make_src_stats():
    mesh = plsc.VectorSubcoreMesh(core_axis_name="c", subcore_axis_name="s")

    @functools.partial(
        pl.kernel, mesh=mesh,
        out_type=jax.ShapeDtypeStruct((_NC, _N, _D), jnp.float32),
        scratch_types=[
            pltpu.VMEM((2 * _CHA,), jnp.int32),
            pltpu.VMEM((2 * _CHA, _D), jnp.float32),
            pltpu.VMEM((2 * _CHA,), jnp.int32),
            pltpu.VMEM((2 * _CHA, _D), jnp.float32),
            pltpu.VMEM((8, _D), jnp.float32),
            pltpu.VMEM_SHARED((_N, _D), jnp.float32),
            pltpu.SemaphoreType.DMA,
            pltpu.SemaphoreType.DMA,
            pltpu.SemaphoreType.DMA,
            pltpu.SemaphoreType.DMA,
        ],
    )
    def k(src_hbm, dst_hbm, e128_hbm, zeros_hbm, consts_hbm, out_hbm, idxva,
          staga, idxvb, stagb, cvbuf, acc, isema, esema, isemb, esemb):
        c = lax.axis_index("c")
        s = lax.axis_index("s")
        wid = c * _NS + s

        pltpu.sync_copy(consts_hbm, cvbuf)

        def irow(i, carry):
            for r in range(_D // 16):
                sl = pl.ds(r * 16, 16)
                staga[_CHA + i, sl] = cvbuf[1, sl]
                stagb[_CHA + i, sl] = cvbuf[1, sl]
            return carry

        lax.fori_loop(0, _CHA, irow, 0)
        pltpu.sync_copy(zeros_hbm.at[pl.ds(s * _RPT, _RPT), :],
                        acc.at[pl.ds(s * _RPT, _RPT), :])

        @pl.when(s == _NS - 1)
        def _():
            pltpu.sync_copy(zeros_hbm.at[pl.ds(_NS * _RPT, 16), :],
                            acc.at[pl.ds(_NS * _RPT, 16), :])

        plsc.subcore_barrier()

        base0 = wid * _EPTA
        abufs = ((idxva, staga, isema, esema), (idxvb, stagb, isemb, esemb))

        def astart(ci, bi):
            idxv, stag, isem, esem = abufs[bi]
            base = base0 + ci * _CHA
            pltpu.async_copy(src_hbm.at[pl.ds(base, _CHA)],
                             idxv.at[pl.ds(0, _CHA)], isem)
            pltpu.async_copy(dst_hbm.at[pl.ds(base, _CHA)],
                             idxv.at[pl.ds(_CHA, _CHA)], isem)
            pltpu.async_copy(e128_hbm.at[pl.ds(base, _CHA), :],
                             stag.at[pl.ds(0, _CHA), :], esem)

        def await_scatter(ci, bi):
            idxv, stag, isem, esem = abufs[bi]
            base = base0 + ci * _CHA
            pltpu.make_async_copy(src_hbm.at[pl.ds(base, _CHA)],
                                  idxv.at[pl.ds(0, _CHA)], isem).wait()
            pltpu.make_async_copy(dst_hbm.at[pl.ds(base, _CHA)],
                                  idxv.at[pl.ds(_CHA, _CHA)], isem).wait()
            pltpu.make_async_copy(e128_hbm.at[pl.ds(base, _CHA), :],
                                  stag.at[pl.ds(0, _CHA), :], esem).wait()
            pltpu.sync_copy(stag, acc.at[idxv], add=True)

        astart(0, 0)
        npaira = _NCHA // 2

        def apair(jj, carry):
            c0 = 2 * jj
            astart(c0 + 1, 1)
            await_scatter(c0, 0)

            @pl.when(jj + 1 < npaira)
            def _():
                astart(c0 + 2, 0)

            await_scatter(c0 + 1, 1)
            return carry

        lax.fori_loop(0, npaira, apair, 0)
        plsc.subcore_barrier()
        pltpu.sync_copy(acc.at[pl.ds(s * _RPT, _RPT), :],
                        out_hbm.at[c, pl.ds(s * _RPT, _RPT), :])

        @pl.when(s == _NS - 1)
        def _():
            pltpu.sync_copy(acc.at[pl.ds(_NS * _RPT, 16), :],
                            out_hbm.at[c, pl.ds(_NS * _RPT, 16), :])

    return k


def _make_edge_pass():
    mesh = plsc.VectorSubcoreMesh(core_axis_name="c", subcore_axis_name="s")

    @functools.partial(
        pl.kernel, mesh=mesh,
        out_type=jax.ShapeDtypeStruct((_NC, _N, _D), jnp.float32),
        scratch_types=[
            pltpu.VMEM((_CHB,), jnp.int32),
            pltpu.VMEM((_CHB,), jnp.int32),
            pltpu.VMEM((_CHB, _D), jnp.float32),
            pltpu.VMEM((_CHB, _D), jnp.float32),
            pltpu.VMEM((_CHB,), jnp.int32),
            pltpu.VMEM((_CHB,), jnp.int32),
            pltpu.VMEM((_CHB, _D), jnp.float32),
            pltpu.VMEM((_CHB, _D), jnp.float32),
            pltpu.VMEM_SHARED((_N, _D), jnp.float32),
            pltpu.SemaphoreType.DMA,
            pltpu.SemaphoreType.DMA,
            pltpu.SemaphoreType.DMA,
            pltpu.SemaphoreType.DMA,
            pltpu.SemaphoreType.DMA,
            pltpu.SemaphoreType.DMA,
        ],
    )
    def k(src_hbm, dst_hbm, xw2_hbm, ew2_hbm, zeros_hbm, out_hbm,
          srcva, dstva, gbufa, wbufa, srcvb, dstvb, gbufb, wbufb, acc,
          isema, esema, gsema, isemb, esemb, gsemb):
        c = lax.axis_index("c")
        s = lax.axis_index("s")

        pltpu.sync_copy(zeros_hbm.at[pl.ds(s * _RPT, _RPT), :],
                        acc.at[pl.ds(s * _RPT, _RPT), :])

        @pl.when(s == _NS - 1)
        def _():
            pltpu.sync_copy(zeros_hbm.at[pl.ds(_NS * _RPT, 16), :],
                            acc.at[pl.ds(_NS * _RPT, 16), :])

        plsc.subcore_barrier()

        base0 = s * _EPTB
        coffv = jnp.full((16,), c * _N, jnp.int32)
        zf16 = jnp.zeros((16,), jnp.float32)
        bufs = ((srcva, dstva, gbufa, wbufa, isema, esema, gsema),
                (srcvb, dstvb, gbufb, wbufb, isemb, esemb, gsemb))

        def start_ie(ci, bi):
            srcv, dstv, _, wbuf, isem, esem, _ = bufs[bi]
            base = base0 + ci * _CHB
            pltpu.async_copy(src_hbm.at[pl.ds(base, _CHB)], srcv, isem)
            pltpu.async_copy(dst_hbm.at[pl.ds(base, _CHB)], dstv, isem)
            pltpu.async_copy(ew2_hbm.at[c, pl.ds(base, _CHB), :], wbuf, esem)

        def wait_idx(ci, bi):
            srcv, dstv, _, _, isem, _, _ = bufs[bi]
            base = base0 + ci * _CHB
            pltpu.make_async_copy(src_hbm.at[pl.ds(base, _CHB)], srcv,
                                  isem).wait()
            pltpu.make_async_copy(dst_hbm.at[pl.ds(base, _CHB)], dstv,
                                  isem).wait()

        def start_gather(bi):
            srcv, _, gbuf, _, _, _, gsem = bufs[bi]
            for b in range(_CHB // 16):
                sl = pl.ds(b * 16, 16)
                srcv[sl] = srcv[sl] + coffv
            pltpu.async_copy(xw2_hbm.at[srcv], gbuf, gsem)

        def compute_scatter(ci, bi):
            srcv, dstv, gbuf, wbuf, _, esem, gsem = bufs[bi]
            base = base0 + ci * _CHB
            pltpu.make_async_copy(xw2_hbm.at[srcv], gbuf, gsem).wait()
            pltpu.make_async_copy(ew2_hbm.at[c, pl.ds(base, _CHB), :], wbuf,
                                  esem).wait()

            def erow(j, carry2):
                for jj in range(2):
                    for r in range(_D // 16):
                        sl = pl.ds(r * 16, 16)
                        a = gbuf[2 * j + jj, sl] + wbuf[2 * j + jj, sl]
                        wbuf[2 * j + jj, sl] = jnp.maximum(a, zf16)
                return carry2

            lax.fori_loop(0, _CHB // 2, erow, 0)
            pltpu.sync_copy(wbuf, acc.at[dstv], add=True)

        start_ie(0, 0)
        start_ie(1, 1)
        wait_idx(0, 0)
        start_gather(0)

        npair = _NCHB // 2

        def pair(jj, carry):
            c0 = 2 * jj
            c1 = c0 + 1
            wait_idx(c1, 1)
            start_gather(1)
            compute_scatter(c0, 0)

            @pl.when(jj + 1 < npair)
            def _():
                start_ie(c0 + 2, 0)
                wait_idx(c0 + 2, 0)
                start_gather(0)

            compute_scatter(c1, 1)

            @pl.when(jj + 1 < npair)
            def _():
                start_ie(c1 + 2, 1)

            return carry

        lax.fori_loop(0, npair, pair, 0)
        plsc.subcore_barrier()
        pltpu.sync_copy(acc.at[pl.ds(s * _RPT, _RPT), :],
                        out_hbm.at[c, pl.ds(s * _RPT, _RPT), :])

        @pl.when(s == _NS - 1)
        def _():
            pltpu.sync_copy(acc.at[pl.ds(_NS * _RPT, 16), :],
                            out_hbm.at[c, pl.ds(_NS * _RPT, 16), :])

    return k


def _finish_body(lo_ref, hi_ref, sa0_ref, sa1_ref, x_ref, w2_ref, b2_ref,
                 o_ref):
    lo = lo_ref[...]
    hi = hi_ref[...]
    agg = jnp.concatenate(
        [lo[:, :_DE], hi, jnp.zeros((lo.shape[0], 4), jnp.float32)], axis=1)
    cnt = (sa0_ref[...] + sa1_ref[...])[:, 5:6]
    h = _dot(agg, w2_ref[...]) + cnt * b2_ref[...]
    o_ref[...] = jnp.where(cnt > 0.5, h, x_ref[...])


def _finish(lo, hi, sa0, sa1, x, w2p, b2r):
    return pl.pallas_call(
        _finish_body,
        grid=(_N // _BN,),
        in_specs=[
            pl.BlockSpec((_BN, _D), lambda i: (i, 0)),
            pl.BlockSpec((_BN, _D), lambda i: (i, 0)),
            pl.BlockSpec((_BN, _D), lambda i: (i, 0)),
            pl.BlockSpec((_BN, _D), lambda i: (i, 0)),
            pl.BlockSpec((_BN, _D), lambda i: (i, 0)),
            pl.BlockSpec((136, _D), lambda i: (0, 0)),
            pl.BlockSpec((1, _D), lambda i: (0, 0)),
        ],
        out_specs=pl.BlockSpec((_BN, _D), lambda i: (i, 0)),
        out_shape=jax.ShapeDtypeStruct((_N, _D), jnp.float32),
    )(lo, hi, sa0, sa1, x, w2p, b2r)


def kernel(x, edge_index, e, W1, b1, gamma, beta, W2, b2):
    src = edge_index[0]
    dst = edge_index[1]
    zeros128 = jnp.zeros((_N, _D), jnp.float32)
    consts = jnp.zeros((8, _D), jnp.float32).at[1, 5].set(1.0)
    e128 = jnp.concatenate(
        [e, jnp.ones((_E, 1), jnp.float32),
         jnp.zeros((_E, _D - _DE - 1), jnp.float32)], axis=1)
    ee = _ee_gram(e)
    srcagg = _make_src_stats()(src, dst, e128, zeros128, consts)
    mx, mct, sv = _stats(x, srcagg[0], srcagg[1])
    fx, fe = _fold(mx, mct, sv, ee, W1, b1.reshape(1, -1),
                   gamma.reshape(1, -1), beta.reshape(1, -1))
    xw2 = _xw(x, fx).reshape(2 * _N, _D)
    ew2 = _ew(e, fe)
    aggw = _make_edge_pass()(src, dst, xw2, ew2, zeros128)
    w2p = jnp.pad(W2, ((0, 4), (0, 0)))
    return _finish(aggw[0], aggw[1], srcagg[0], srcagg[1], x, w2p,
                   b2.reshape(1, -1))
